# Initial kernel scaffold; baseline (speedup 1.0000x reference)
import jax
import jax.numpy as jnp
from jax.experimental import pallas as pl


def kernel(pc, feats, W_res, b_res, W_0, b_0, W_l0, b_l0, W_l1, b_l1, W_s0, b_s0, W_f0, b_f0, W_s1, b_s1, W_f1, b_f1, W_1, b_1):
    raise NotImplementedError("write your pallas kernel here")


# speedup vs baseline: 13.2397x; 13.2397x over previous
import functools

import jax
import jax.numpy as jnp
from jax import lax
from jax.experimental import pallas as pl
from jax.experimental.pallas import tpu as pltpu
from jax.experimental.pallas import tpu_sc as plsc

_K = 16
_TW = 128
_TP1 = 256
_TP2 = 1024
_NC, _NS = 2, 16
_CH = 128


def _stage1_body(feats_ref, pc_ref, pcT_ref, pc2_ref, w0_ref, b0_ref,
                 table_ref, idx_ref, d_ref):
    b = pl.program_id(0)
    n = pc_ref.shape[1]

    x0 = jnp.maximum(feats_ref[...] @ w0_ref[...] + b0_ref[...], 0.0)
    pad = jnp.zeros((x0.shape[0], _TW - x0.shape[1] - 2), jnp.float32)
    table_ref[...] = jnp.concatenate([x0, pc2_ref[...], pad], axis=1)

    pc_all = pc_ref[0]
    pct = pcT_ref[0]
    xc, yc = pc_all[:, 0:1], pc_all[:, 1:2]
    xt, yt = pct[0:1, :], pct[1:2, :]
    sqc = xc * xc + yc * yc
    sqt = xt * xt + yt * yt
    dot = jnp.dot(pc_all, pct, precision=lax.Precision.DEFAULT)
    d_ref[...] = (sqt - 2.0 * dot) + sqc

    iota = lax.broadcasted_iota(jnp.int32, (n, _TP1), 0)
    rows = []
    for _ in range(_K):
        d = d_ref[...]
        m = jnp.min(d, axis=0, keepdims=True)
        sel = jnp.where(d == m, iota, n)
        idxv = jnp.min(sel, axis=0, keepdims=True)
        rows.append(idxv + b * n)
        d_ref[...] = jnp.where(iota == idxv, jnp.inf, d)
    idx_ref[0] = jnp.concatenate(rows, axis=0)


def _stage1(feats2d, pc, pcT, pc2d, W_0, b_0):
    bsz, n, _ = pc.shape
    nt = n // _TP1
    f = feats2d.shape[1]
    u = W_0.shape[1]
    return pl.pallas_call(
        _stage1_body,
        grid=(bsz, nt),
        in_specs=[
            pl.BlockSpec((_TP1, f), lambda b, t: (b * nt + t, 0)),
            pl.BlockSpec((1, n, 2), lambda b, t: (b, 0, 0)),
            pl.BlockSpec((1, 2, _TP1), lambda b, t: (b, 0, t)),
            pl.BlockSpec((_TP1, 2), lambda b, t: (b * nt + t, 0)),
            pl.BlockSpec((f, u), lambda b, t: (0, 0)),
            pl.BlockSpec((1, u), lambda b, t: (0, 0)),
        ],
        out_specs=[
            pl.BlockSpec((_TP1, _TW), lambda b, t: (b * nt + t, 0)),
            pl.BlockSpec((1, _K, _TP1), lambda b, t: (b, 0, t)),
        ],
        out_shape=[
            jax.ShapeDtypeStruct((bsz * n, _TW), jnp.float32),
            jax.ShapeDtypeStruct((bsz, _K, n), jnp.int32),
        ],
        scratch_shapes=[pltpu.VMEM((n, _TP1), jnp.float32)],
    )(feats2d, pc, pcT, pc2d, W_0, b_0)


def _sc_gather(idx_flat, table):
    tot = idx_flat.shape[0]
    nw = _NC * _NS
    span = tot // nw
    nch = span // _CH
    mesh = plsc.VectorSubcoreMesh(core_axis_name="c", subcore_axis_name="s")

    @functools.partial(
        pl.kernel,
        out_type=jax.ShapeDtypeStruct((tot, _TW), jnp.float32),
        mesh=mesh,
        scratch_types=[
            pltpu.VMEM((_CH,), jnp.int32),
            pltpu.VMEM((_CH, _TW), jnp.float32),
            pltpu.SemaphoreType.DMA,
        ],
    )
    def gk(idx_hbm, table_hbm, out_hbm, idx_v, rows_v, sem):
        wid = lax.axis_index("s") * _NC + lax.axis_index("c")
        base0 = wid * span

        @pl.loop(0, nch)
        def _chunk(c):
            base = base0 + c * _CH
            pltpu.sync_copy(idx_hbm.at[pl.ds(base, _CH)], idx_v)
            pltpu.async_copy(table_hbm.at[idx_v], rows_v, sem).wait()
            pltpu.sync_copy(rows_v, out_hbm.at[pl.ds(base, _CH)])

    return gk(idx_flat, table)


def _stage2_body(g_ref, pc2_ref, feats_ref, wl1_ref, bl1_ref, ws1_ref,
                 bs1_ref, wf1_ref, bf1_ref, w1_ref, b1_ref, wres_ref,
                 bres_ref, out_ref, acc_ref):
    k = pl.program_id(2)

    @pl.when(k == 0)
    def _():
        acc_ref[...] = jnp.zeros_like(acc_ref)

    g = g_ref[...]
    xj = g[:, 0:32]
    pj = g[:, 32:34]
    pi = pc2_ref[...]
    rel = pi - pj
    norm = jnp.sqrt(rel[:, 0:1] * rel[:, 0:1]
                    + rel[:, 1:2] * rel[:, 1:2] + 1e-12)
    rppe = jnp.concatenate([pi, pj, rel, norm], axis=1)
    r1 = jnp.maximum(rppe @ wl1_ref[...] + bl1_ref[...], 0.0)
    l1 = jnp.concatenate([xj, r1], axis=1)
    t = l1 @ ws1_ref[...] + bs1_ref[...]
    e = jnp.exp(t - jnp.max(t, axis=1, keepdims=True))
    s = e / jnp.sum(e, axis=1, keepdims=True)
    acc_ref[...] += l1 * s

    @pl.when(k == _K - 1)
    def _():
        a1 = acc_ref[...]
        x2 = jnp.maximum(a1 @ wf1_ref[...] + bf1_ref[...], 0.0)
        x3 = jnp.maximum(x2 @ w1_ref[...] + b1_ref[...], 0.0)
        y = jnp.maximum(feats_ref[...] @ wres_ref[...] + bres_ref[...], 0.0)
        z = x3 + y
        out_ref[...] = jnp.where(z > 0.0, z, 0.2 * z)


def _stage2(g, pc2d, feats2d, W_l1, b_l1, W_s1, b_s1, W_f1, b_f1, W_1, b_1,
            W_res, b_res, bsz, n):
    nt = n // _TP2
    f = feats2d.shape[1]
    u = W_1.shape[1]

    def wspec(w):
        return pl.BlockSpec(w.shape, lambda b, t, k: (0, 0))

    return pl.pallas_call(
        _stage2_body,
        grid=(bsz, nt, _K),
        in_specs=[
            pl.BlockSpec((_TP2, _TW), lambda b, t, k: ((b * _K + k) * nt + t, 0)),
            pl.BlockSpec((_TP2, 2), lambda b, t, k: (b * nt + t, 0)),
            pl.BlockSpec((_TP2, f), lambda b, t, k: (b * nt + t, 0)),
            wspec(W_l1), wspec(b_l1), wspec(W_s1), wspec(b_s1),
            wspec(W_f1), wspec(b_f1), wspec(W_1), wspec(b_1),
            wspec(W_res), wspec(b_res),
        ],
        out_specs=pl.BlockSpec((_TP2, u), lambda b, t, k: (b * nt + t, 0)),
        out_shape=jax.ShapeDtypeStruct((bsz * n, u), jnp.float32),
        scratch_shapes=[pltpu.VMEM((_TP2, 64), jnp.float32)],
    )(g, pc2d, feats2d, W_l1, b_l1, W_s1, b_s1, W_f1, b_f1, W_1, b_1,
      W_res, b_res)


def kernel(pc, feats, W_res, b_res, W_0, b_0, W_l0, b_l0, W_l1, b_l1,
           W_s0, b_s0, W_f0, b_f0, W_s1, b_s1, W_f1, b_f1, W_1, b_1):
    bsz, n, _ = pc.shape
    f = feats.shape[-1]

    feats2d = feats.reshape(bsz * n, f)
    pc2d = pc.reshape(bsz * n, 2)
    pcT = jnp.transpose(pc, (0, 2, 1))

    table, idxg = _stage1(feats2d, pc, pcT, pc2d, W_0, b_0.reshape(1, -1))
    g = _sc_gather(idxg.reshape(bsz * _K * n), table)
    out = _stage2(g, pc2d, feats2d,
                  W_l1, b_l1.reshape(1, -1), W_s1, b_s1.reshape(1, -1),
                  W_f1, b_f1.reshape(1, -1), W_1, b_1.reshape(1, -1),
                  W_res, b_res.reshape(1, -1), bsz, n)
    return out.reshape(bsz, n, W_1.shape[1])
